# baseline (speedup 1.0000x reference)
import jax
import jax.numpy as jnp
from jax import lax
from jax.experimental import pallas as pl
from jax.experimental.pallas import tpu as pltpu
from jax.experimental.pallas import tpu_sc as plsc

HOURS = 24 * 7
SU, SL, TU, TL = 1000.0, 0.0, 500.0, 0.0
B, L, D = 1024, 20, 64
V = 100000
N = B * L
NW = 32
ROWS_W = N // NW
CHUNK = 128
NCH = ROWS_W // CHUNK
DEPTH = 3


CR = 4000


def _concat_body(l_ref, u_ref, o_ref):
    o_ref[:, 0:D] = l_ref[...]
    o_ref[:, D:2 * D] = u_ref[...]


def _concat(emb_l, emb_u):
    return pl.pallas_call(
        _concat_body,
        grid=(V // CR,),
        in_specs=[
            pl.BlockSpec((CR, D), lambda i: (i, 0)),
            pl.BlockSpec((CR, D), lambda i: (i, 0)),
        ],
        out_specs=pl.BlockSpec((CR, 2 * D), lambda i: (i, 0)),
        out_shape=jax.ShapeDtypeStruct((V, 2 * D), jnp.float32),
    )(emb_l, emb_u)


def _gather_sc(emb_lu_hbm, idx_l_hbm, idx_u_hbm, out_l_hbm, out_u_hbm,
               idx_l_v, idx_u_v, gl, gu, sem_g, sem_w):
    wid = lax.axis_index("s") * 2 + lax.axis_index("c")
    base = wid * ROWS_W

    pltpu.sync_copy(idx_l_hbm.at[wid], idx_l_v)
    pltpu.sync_copy(idx_u_hbm.at[wid], idx_u_v)

    def g_copies(c):
        s = c % DEPTH
        return (pltpu.make_async_copy(emb_lu_hbm.at[idx_l_v.at[c]],
                                      gl.at[s], sem_g),
                pltpu.make_async_copy(emb_lu_hbm.at[idx_u_v.at[c]],
                                      gu.at[s], sem_g))

    def w_copies(c):
        s = c % DEPTH
        r = pl.ds(base + c * CHUNK, CHUNK)
        return (pltpu.make_async_copy(gl.at[s], out_l_hbm.at[r], sem_w),
                pltpu.make_async_copy(gu.at[s], out_u_hbm.at[r], sem_w))

    for c in range(DEPTH):
        for cp in g_copies(c):
            cp.start()
    for c in range(NCH):
        for cp in g_copies(c):
            cp.wait()
        for cp in w_copies(c):
            cp.start()
        if c + DEPTH < NCH:
            for cp in w_copies(c):
                cp.wait()
            for cp in g_copies(c + DEPTH):
                cp.start()
    for c in range(max(0, NCH - DEPTH), NCH):
        for cp in w_copies(c):
            cp.wait()


def _gather(emb_lu, idx_l, idx_u):
    mesh = plsc.VectorSubcoreMesh(core_axis_name="c", subcore_axis_name="s")
    return pl.kernel(
        _gather_sc,
        out_type=(jax.ShapeDtypeStruct((N, 2 * D), jnp.float32),
                  jax.ShapeDtypeStruct((N, 2 * D), jnp.float32)),
        mesh=mesh,
        compiler_params=pltpu.CompilerParams(use_tc_tiling_on_sc=False),
        scratch_types=[
            pltpu.VMEM((8, CHUNK), jnp.int32),
            pltpu.VMEM((8, CHUNK), jnp.int32),
            pltpu.VMEM((DEPTH, CHUNK, 2 * D), jnp.float32),
            pltpu.VMEM((DEPTH, CHUNK, 2 * D), jnp.float32),
            pltpu.SemaphoreType.DMA,
            pltpu.SemaphoreType.DMA,
        ],
    )(emb_lu, idx_l, idx_u)


BJ = 32


def _joint_body(tf_ref, g_l_ref, g_u_ref, et_ref, out_ref):
    et = et_ref[...].astype(jnp.bfloat16)
    iot = lax.broadcasted_iota(jnp.int32, (HOURS + 1, 1), 0)
    oh = (iot == tf_ref[0]).astype(jnp.bfloat16)
    tpart = lax.dot_general(oh, et, (((0,), (0,)), ((), ())),
                            preferred_element_type=jnp.float32)
    out_ref[...] = g_l_ref[:, 0:D] + g_u_ref[:, D:2 * D] + tpart


def _joint(tf, g_l, g_u, emb_t):
    grid = (B // BJ,)
    gspec = pl.BlockSpec((BJ * L, 2 * D), lambda i: (i, 0))
    return pl.pallas_call(
        _joint_body,
        grid=grid,
        in_specs=[
            pl.BlockSpec((1, 1, BJ * L), lambda i: (i, 0, 0)),
            gspec, gspec,
            pl.BlockSpec((HOURS + 1, D), lambda i: (0, 0)),
        ],
        out_specs=pl.BlockSpec((BJ * L, D), lambda i: (i, 0)),
        out_shape=jax.ShapeDtypeStruct((N, D), jnp.float32),
    )(tf, g_l, g_u, emb_t)


BB = 32
Q = L * L // 2


def _delta_body(lens_ref, dsE_ref, dtE_ref, dsO_ref, dtO_ref,
                su_ref, sl_ref, tu_ref, tl_ref, out_ref):
    sl = sl_ref[...]
    su = su_ref[...]
    tl = tl_ref[...]
    tu = tu_ref[...]
    a0 = (sl[0] + tl[0])[None, :]
    da = (sl[1] + tl[1])[None, :] - a0
    b0 = ((su[0] - sl[0]) * (1.0 / (SU - SL)))[None, :]
    db = ((su[1] - sl[1]) * (1.0 / (SU - SL)))[None, :] - b0
    c0 = ((tu[0] - tl[0]) * (1.0 / (TU - TL)))[None, :]
    dc = ((tu[1] - tl[1]) * (1.0 / (TU - TL)))[None, :] - c0
    zz = jnp.zeros((2, D), jnp.float32)
    vtop = jnp.concatenate([a0, b0, c0, da, db, dc, zz], axis=0)
    z8 = jnp.zeros((8, D), jnp.float32)
    v2 = jnp.concatenate(
        [jnp.concatenate([vtop, z8], axis=1),
         jnp.concatenate([z8, vtop], axis=1)], axis=0
    ).astype(jnp.bfloat16)

    q = lax.broadcasted_iota(jnp.int32, (1, Q), 1)
    pe = 2 * q
    po = 2 * q + 1
    ie, je = pe // L, pe % L
    io, jo = po // L, po % L
    ones = jnp.ones((1, Q), jnp.float32)
    zrow = jnp.zeros((2, Q), jnp.float32)
    lens = lens_ref[...]
    for b in range(BB):
        lb = lens[b:b + 1, 0:1]
        me = ((ie < lb) & (je < lb)).astype(jnp.float32)
        mo = ((io < lb) & (jo < lb)).astype(jnp.float32)
        dse = dsE_ref[b:b + 1, :]
        dte = dtE_ref[b:b + 1, :]
        dso = dsO_ref[b:b + 1, :]
        dto = dtO_ref[b:b + 1, :]
        ct = jnp.concatenate(
            [ones, dse, dte, me, me * dse, me * dte, zrow,
             ones, dso, dto, mo, mo * dso, mo * dto, zrow], axis=0
        ).astype(jnp.bfloat16)
        out_ref[b] = lax.dot_general(
            ct, v2, (((0,), (0,)), ((), ())),
            preferred_element_type=jnp.float32)


def _delta(lens, dsE, dtE, dsO, dtO, emb_su, emb_sl, emb_tu, emb_tl):
    grid = (B // BB,)
    small = pl.BlockSpec((2, D), lambda i: (0, 0))
    coef = pl.BlockSpec((BB, Q), lambda i: (i, 0))
    return pl.pallas_call(
        _delta_body,
        grid=grid,
        in_specs=[
            pl.BlockSpec((BB, 1), lambda i: (i, 0)),
            coef, coef, coef, coef,
            small, small, small, small,
        ],
        out_specs=pl.BlockSpec((BB, Q, 2 * D), lambda i: (i, 0, 0)),
        out_shape=jax.ShapeDtypeStruct((B, Q, 2 * D), jnp.float32),
    )(lens, dsE, dtE, dsO, dtO, emb_su, emb_sl, emb_tu, emb_tl)


def kernel(traj, mat, traj_len, emb_t, emb_l, emb_u, emb_su, emb_sl,
           emb_tu, emb_tl):
    emb_lu = _concat(emb_l, emb_u)
    idx_l = jnp.pad(traj[:, :, 1].reshape(NW, NCH, CHUNK),
                    ((0, 0), (0, 8 - NCH), (0, 0)))
    idx_u = jnp.pad(traj[:, :, 0].reshape(NW, NCH, CHUNK),
                    ((0, 0), (0, 8 - NCH), (0, 0)))
    g_l, g_u = _gather(emb_lu, idx_l, idx_u)

    t_idx = (traj[:, :, 2] - 1) % HOURS + 1
    tf = t_idx.reshape(B // BJ, 1, BJ * L)
    joint = _joint(tf, g_l, g_u, emb_t).reshape(B, L, D)

    m800 = mat.reshape(B, 2 * L * L)
    dsE = m800[:, 0::4]
    dtE = m800[:, 1::4]
    dsO = m800[:, 2::4]
    dtO = m800[:, 3::4]
    delta = _delta(traj_len.reshape(B, 1), dsE, dtE, dsO, dtO, emb_su,
                   emb_sl, emb_tu, emb_tl).reshape(B, L, L, D)
    return (joint, delta)


# speedup vs baseline: 1.0016x; 1.0016x over previous
import jax
import jax.numpy as jnp
from jax import lax
from jax.experimental import pallas as pl
from jax.experimental.pallas import tpu as pltpu
from jax.experimental.pallas import tpu_sc as plsc

HOURS = 24 * 7
SU, SL, TU, TL = 1000.0, 0.0, 500.0, 0.0
B, L, D = 1024, 20, 64
V = 100000
N = B * L
NW = 32
ROWS_W = N // NW
CHUNK = 128
NCH = ROWS_W // CHUNK
DEPTH = 3


CR = 4000


def _concat_body(l_ref, u_ref, o_ref):
    o_ref[:, 0:D] = l_ref[...]
    o_ref[:, D:2 * D] = u_ref[...]


def _concat(emb_l, emb_u):
    return pl.pallas_call(
        _concat_body,
        grid=(V // CR,),
        in_specs=[
            pl.BlockSpec((CR, D), lambda i: (i, 0)),
            pl.BlockSpec((CR, D), lambda i: (i, 0)),
        ],
        out_specs=pl.BlockSpec((CR, 2 * D), lambda i: (i, 0)),
        out_shape=jax.ShapeDtypeStruct((V, 2 * D), jnp.float32),
    )(emb_l, emb_u)


def _gather_sc(emb_lu_hbm, idx_l_hbm, idx_u_hbm, out_l_hbm, out_u_hbm,
               idx_l_v, idx_u_v, gl, gu, sem_g, sem_w):
    wid = lax.axis_index("s") * 2 + lax.axis_index("c")
    base = wid * ROWS_W

    pltpu.sync_copy(idx_l_hbm.at[wid], idx_l_v)
    pltpu.sync_copy(idx_u_hbm.at[wid], idx_u_v)

    def g_copies(c):
        s = c % DEPTH
        return (pltpu.make_async_copy(emb_lu_hbm.at[idx_l_v.at[c]],
                                      gl.at[s], sem_g),
                pltpu.make_async_copy(emb_lu_hbm.at[idx_u_v.at[c]],
                                      gu.at[s], sem_g))

    def w_copies(c):
        s = c % DEPTH
        r = pl.ds(base + c * CHUNK, CHUNK)
        return (pltpu.make_async_copy(gl.at[s], out_l_hbm.at[r], sem_w),
                pltpu.make_async_copy(gu.at[s], out_u_hbm.at[r], sem_w))

    for c in range(DEPTH):
        for cp in g_copies(c):
            cp.start()
    for c in range(NCH):
        for cp in g_copies(c):
            cp.wait()
        for cp in w_copies(c):
            cp.start()
        if c + DEPTH < NCH:
            for cp in w_copies(c):
                cp.wait()
            for cp in g_copies(c + DEPTH):
                cp.start()
    for c in range(max(0, NCH - DEPTH), NCH):
        for cp in w_copies(c):
            cp.wait()


def _gather(emb_lu, idx_l, idx_u):
    mesh = plsc.VectorSubcoreMesh(core_axis_name="c", subcore_axis_name="s")
    return pl.kernel(
        _gather_sc,
        out_type=(jax.ShapeDtypeStruct((N, 2 * D), jnp.float32),
                  jax.ShapeDtypeStruct((N, 2 * D), jnp.float32)),
        mesh=mesh,
        scratch_types=[
            pltpu.VMEM((8, CHUNK), jnp.int32),
            pltpu.VMEM((8, CHUNK), jnp.int32),
            pltpu.VMEM((DEPTH, CHUNK, 2 * D), jnp.float32),
            pltpu.VMEM((DEPTH, CHUNK, 2 * D), jnp.float32),
            pltpu.SemaphoreType.DMA,
            pltpu.SemaphoreType.DMA,
        ],
    )(emb_lu, idx_l, idx_u)


BJ = 32


def _joint_body(tf_ref, g_l_ref, g_u_ref, et_ref, out_ref):
    et = et_ref[...].astype(jnp.bfloat16)
    iot = lax.broadcasted_iota(jnp.int32, (HOURS + 1, 1), 0)
    oh = (iot == tf_ref[0]).astype(jnp.bfloat16)
    tpart = lax.dot_general(oh, et, (((0,), (0,)), ((), ())),
                            preferred_element_type=jnp.float32)
    out_ref[...] = g_l_ref[:, 0:D] + g_u_ref[:, D:2 * D] + tpart


def _joint(tf, g_l, g_u, emb_t):
    grid = (B // BJ,)
    gspec = pl.BlockSpec((BJ * L, 2 * D), lambda i: (i, 0))
    return pl.pallas_call(
        _joint_body,
        grid=grid,
        in_specs=[
            pl.BlockSpec((1, 1, BJ * L), lambda i: (i, 0, 0)),
            gspec, gspec,
            pl.BlockSpec((HOURS + 1, D), lambda i: (0, 0)),
        ],
        out_specs=pl.BlockSpec((BJ * L, D), lambda i: (i, 0)),
        out_shape=jax.ShapeDtypeStruct((N, D), jnp.float32),
    )(tf, g_l, g_u, emb_t)


BB = 32
Q = L * L // 2


def _delta_body(lens_ref, dsE_ref, dtE_ref, dsO_ref, dtO_ref,
                su_ref, sl_ref, tu_ref, tl_ref, out_ref):
    sl = sl_ref[...]
    su = su_ref[...]
    tl = tl_ref[...]
    tu = tu_ref[...]
    a0 = (sl[0] + tl[0])[None, :]
    da = (sl[1] + tl[1])[None, :] - a0
    b0 = ((su[0] - sl[0]) * (1.0 / (SU - SL)))[None, :]
    db = ((su[1] - sl[1]) * (1.0 / (SU - SL)))[None, :] - b0
    c0 = ((tu[0] - tl[0]) * (1.0 / (TU - TL)))[None, :]
    dc = ((tu[1] - tl[1]) * (1.0 / (TU - TL)))[None, :] - c0
    zz = jnp.zeros((2, D), jnp.float32)
    vtop = jnp.concatenate([a0, b0, c0, da, db, dc, zz], axis=0)
    z8 = jnp.zeros((8, D), jnp.float32)
    v2 = jnp.concatenate(
        [jnp.concatenate([vtop, z8], axis=1),
         jnp.concatenate([z8, vtop], axis=1)], axis=0
    ).astype(jnp.bfloat16)

    q = lax.broadcasted_iota(jnp.int32, (1, Q), 1)
    pe = 2 * q
    po = 2 * q + 1
    ie, je = pe // L, pe % L
    io, jo = po // L, po % L
    ones = jnp.ones((1, Q), jnp.float32)
    zrow = jnp.zeros((2, Q), jnp.float32)
    lens = lens_ref[...]
    for b in range(BB):
        lb = lens[b:b + 1, 0:1]
        me = ((ie < lb) & (je < lb)).astype(jnp.float32)
        mo = ((io < lb) & (jo < lb)).astype(jnp.float32)
        dse = dsE_ref[b:b + 1, :]
        dte = dtE_ref[b:b + 1, :]
        dso = dsO_ref[b:b + 1, :]
        dto = dtO_ref[b:b + 1, :]
        ct = jnp.concatenate(
            [ones, dse, dte, me, me * dse, me * dte, zrow,
             ones, dso, dto, mo, mo * dso, mo * dto, zrow], axis=0
        ).astype(jnp.bfloat16)
        out_ref[b] = lax.dot_general(
            ct, v2, (((0,), (0,)), ((), ())),
            preferred_element_type=jnp.float32)


def _delta(lens, dsE, dtE, dsO, dtO, emb_su, emb_sl, emb_tu, emb_tl):
    grid = (B // BB,)
    small = pl.BlockSpec((2, D), lambda i: (0, 0))
    coef = pl.BlockSpec((BB, Q), lambda i: (i, 0))
    return pl.pallas_call(
        _delta_body,
        grid=grid,
        in_specs=[
            pl.BlockSpec((BB, 1), lambda i: (i, 0)),
            coef, coef, coef, coef,
            small, small, small, small,
        ],
        out_specs=pl.BlockSpec((BB, Q, 2 * D), lambda i: (i, 0, 0)),
        out_shape=jax.ShapeDtypeStruct((B, Q, 2 * D), jnp.float32),
    )(lens, dsE, dtE, dsO, dtO, emb_su, emb_sl, emb_tu, emb_tl)


def kernel(traj, mat, traj_len, emb_t, emb_l, emb_u, emb_su, emb_sl,
           emb_tu, emb_tl):
    emb_lu = _concat(emb_l, emb_u)
    idx_l = jnp.pad(traj[:, :, 1].reshape(NW, NCH, CHUNK),
                    ((0, 0), (0, 8 - NCH), (0, 0)))
    idx_u = jnp.pad(traj[:, :, 0].reshape(NW, NCH, CHUNK),
                    ((0, 0), (0, 8 - NCH), (0, 0)))
    g_l, g_u = _gather(emb_lu, idx_l, idx_u)

    t_idx = (traj[:, :, 2] - 1) % HOURS + 1
    tf = t_idx.reshape(B // BJ, 1, BJ * L)
    joint = _joint(tf, g_l, g_u, emb_t).reshape(B, L, D)

    m800 = mat.reshape(B, 2 * L * L)
    dsE = m800[:, 0::4]
    dtE = m800[:, 1::4]
    dsO = m800[:, 2::4]
    dtO = m800[:, 3::4]
    delta = _delta(traj_len.reshape(B, 1), dsE, dtE, dsO, dtO, emb_su,
                   emb_sl, emb_tu, emb_tl).reshape(B, L, L, D)
    return (joint, delta)


# speedup vs baseline: 1.7207x; 1.7180x over previous
import jax
import jax.numpy as jnp
from jax import lax
from jax.experimental import pallas as pl
from jax.experimental.pallas import tpu as pltpu
from jax.experimental.pallas import tpu_sc as plsc

HOURS = 24 * 7
SU, SL, TU, TL = 1000.0, 0.0, 500.0, 0.0
B, L, D = 1024, 20, 64
V = 100000
N = B * L
NW = 32
ROWS_W = N // NW
CHUNK = 128
NCH = ROWS_W // CHUNK
DEPTH = 3


CR = 6400


def _concat_body(lT_ref, uT_ref, o_ref):
    o_ref[:, 0:D] = lT_ref[...].T
    o_ref[:, D:2 * D] = uT_ref[...].T


def _concat(emb_lT, emb_uT):
    return pl.pallas_call(
        _concat_body,
        grid=(pl.cdiv(V, CR),),
        in_specs=[
            pl.BlockSpec((D, CR), lambda i: (0, i)),
            pl.BlockSpec((D, CR), lambda i: (0, i)),
        ],
        out_specs=pl.BlockSpec((CR, 2 * D), lambda i: (i, 0)),
        out_shape=jax.ShapeDtypeStruct((V, 2 * D), jnp.float32),
    )(emb_lT, emb_uT)


def _gather_sc(emb_lu_hbm, idx_l_hbm, idx_u_hbm, out_l_hbm, out_u_hbm,
               idx_l_v, idx_u_v, gl, gu, sem_g, sem_w):
    wid = lax.axis_index("s") * 2 + lax.axis_index("c")
    base = wid * ROWS_W

    pltpu.sync_copy(idx_l_hbm.at[wid], idx_l_v)
    pltpu.sync_copy(idx_u_hbm.at[wid], idx_u_v)

    def g_copies(c):
        s = c % DEPTH
        return (pltpu.make_async_copy(emb_lu_hbm.at[idx_l_v.at[c]],
                                      gl.at[s], sem_g),
                pltpu.make_async_copy(emb_lu_hbm.at[idx_u_v.at[c]],
                                      gu.at[s], sem_g))

    def w_copies(c):
        s = c % DEPTH
        r = pl.ds(base + c * CHUNK, CHUNK)
        return (pltpu.make_async_copy(gl.at[s], out_l_hbm.at[r], sem_w),
                pltpu.make_async_copy(gu.at[s], out_u_hbm.at[r], sem_w))

    for c in range(DEPTH):
        for cp in g_copies(c):
            cp.start()
    for c in range(NCH):
        for cp in g_copies(c):
            cp.wait()
        for cp in w_copies(c):
            cp.start()
        if c + DEPTH < NCH:
            for cp in w_copies(c):
                cp.wait()
            for cp in g_copies(c + DEPTH):
                cp.start()
    for c in range(max(0, NCH - DEPTH), NCH):
        for cp in w_copies(c):
            cp.wait()


def _gather(emb_lu, idx_l, idx_u):
    mesh = plsc.VectorSubcoreMesh(core_axis_name="c", subcore_axis_name="s")
    return pl.kernel(
        _gather_sc,
        out_type=(jax.ShapeDtypeStruct((N, 2 * D), jnp.float32),
                  jax.ShapeDtypeStruct((N, 2 * D), jnp.float32)),
        mesh=mesh,
        scratch_types=[
            pltpu.VMEM((8, CHUNK), jnp.int32),
            pltpu.VMEM((8, CHUNK), jnp.int32),
            pltpu.VMEM((DEPTH, CHUNK, 2 * D), jnp.float32),
            pltpu.VMEM((DEPTH, CHUNK, 2 * D), jnp.float32),
            pltpu.SemaphoreType.DMA,
            pltpu.SemaphoreType.DMA,
        ],
    )(emb_lu, idx_l, idx_u)


BJ = 32


def _joint_body(tf_ref, g_l_ref, g_u_ref, et_ref, out_ref):
    et = et_ref[...].astype(jnp.bfloat16)
    iot = lax.broadcasted_iota(jnp.int32, (HOURS + 1, 1), 0)
    oh = (iot == tf_ref[0]).astype(jnp.bfloat16)
    tpart = lax.dot_general(oh, et, (((0,), (0,)), ((), ())),
                            preferred_element_type=jnp.float32)
    out_ref[...] = g_l_ref[:, 0:D] + g_u_ref[:, D:2 * D] + tpart


def _joint(tf, g_l, g_u, emb_t):
    grid = (B // BJ,)
    gspec = pl.BlockSpec((BJ * L, 2 * D), lambda i: (i, 0))
    return pl.pallas_call(
        _joint_body,
        grid=grid,
        in_specs=[
            pl.BlockSpec((1, 1, BJ * L), lambda i: (i, 0, 0)),
            gspec, gspec,
            pl.BlockSpec((HOURS + 1, D), lambda i: (0, 0)),
        ],
        out_specs=pl.BlockSpec((BJ * L, D), lambda i: (i, 0)),
        out_shape=jax.ShapeDtypeStruct((N, D), jnp.float32),
    )(tf, g_l, g_u, emb_t)


def _delta_body(lens_ref, coef_ref, su_ref, sl_ref, tu_ref, tl_ref,
                out_ref):
    sl = sl_ref[...]
    su = su_ref[...]
    tl = tl_ref[...]
    tu = tu_ref[...]
    a0 = (sl[0] + tl[0])[None, :]
    da = (sl[1] + tl[1])[None, :] - a0
    b0 = ((su[0] - sl[0]) * (1.0 / (SU - SL)))[None, :]
    db = ((su[1] - sl[1]) * (1.0 / (SU - SL)))[None, :] - b0
    c0 = ((tu[0] - tl[0]) * (1.0 / (TU - TL)))[None, :]
    dc = ((tu[1] - tl[1]) * (1.0 / (TU - TL)))[None, :] - c0
    zz = jnp.zeros((2, D), jnp.float32)
    vtop = jnp.concatenate([a0, b0, c0, da, db, dc, zz],
                           axis=0).astype(jnp.bfloat16)

    g = pl.program_id(0)
    lens = lens_ref[...]
    blk = coef_ref[...]
    ones = jnp.ones((1, B), jnp.float32)
    zrow = jnp.zeros((2, B), jnp.float32)
    for t in range(4):
        p = 4 * g + t
        i = p // L
        j = p % L
        m = ((i < lens) & (j < lens)).astype(jnp.float32)
        dsr = blk[2 * t:2 * t + 1]
        dtr = blk[2 * t + 1:2 * t + 2]
        ct = jnp.concatenate(
            [ones, dsr, dtr, m, m * dsr, m * dtr, zrow], axis=0
        ).astype(jnp.bfloat16)
        out_ref[t] = lax.dot_general(
            vtop, ct, (((0,), (0,)), ((), ())),
            preferred_element_type=jnp.float32)


def _delta(lens, m800T, emb_su, emb_sl, emb_tu, emb_tl):
    grid = (L * L // 4,)
    small = pl.BlockSpec((2, D), lambda i: (0, 0))
    return pl.pallas_call(
        _delta_body,
        grid=grid,
        in_specs=[
            pl.BlockSpec((1, B), lambda i: (0, 0)),
            pl.BlockSpec((8, B), lambda i: (i, 0)),
            small, small, small, small,
        ],
        out_specs=pl.BlockSpec((4, D, B), lambda i: (i, 0, 0)),
        out_shape=jax.ShapeDtypeStruct((L * L, D, B), jnp.float32),
    )(lens, m800T, emb_su, emb_sl, emb_tu, emb_tl)


def kernel(traj, mat, traj_len, emb_t, emb_l, emb_u, emb_su, emb_sl,
           emb_tu, emb_tl):
    emb_lu = _concat(emb_l.T, emb_u.T)
    idx_l = jnp.pad(traj[:, :, 1].reshape(NW, NCH, CHUNK),
                    ((0, 0), (0, 8 - NCH), (0, 0)))
    idx_u = jnp.pad(traj[:, :, 0].reshape(NW, NCH, CHUNK),
                    ((0, 0), (0, 8 - NCH), (0, 0)))
    g_l, g_u = _gather(emb_lu, idx_l, idx_u)

    t_idx = (traj[:, :, 2] - 1) % HOURS + 1
    tf = t_idx.reshape(B // BJ, 1, BJ * L)
    joint = _joint(tf, g_l, g_u, emb_t).reshape(B, L, D)

    m800T = mat.transpose(1, 2, 3, 0).reshape(2 * L * L, B)
    delta_t = _delta(traj_len.reshape(1, B), m800T, emb_su, emb_sl,
                     emb_tu, emb_tl)
    delta = delta_t.reshape(L, L, D, B).transpose(3, 0, 1, 2)
    return (joint, delta)


# speedup vs baseline: 2.1091x; 1.2257x over previous
import jax
import jax.numpy as jnp
from jax import lax
from jax.experimental import pallas as pl
from jax.experimental.pallas import tpu as pltpu
from jax.experimental.pallas import tpu_sc as plsc

HOURS = 24 * 7
SU, SL, TU, TL = 1000.0, 0.0, 500.0, 0.0
B, L, D = 1024, 20, 64
V = 100000
N = B * L
NW = 32
ROWS_W = N // NW
CHUNK = 128
NCH = ROWS_W // CHUNK
DEPTH = 3


CR = 6400


def _concat_body(lT_ref, uT_ref, o_ref):
    o_ref[:, 0:D] = lT_ref[...].T
    o_ref[:, D:2 * D] = uT_ref[...].T


def _concat(emb_lT, emb_uT):
    return pl.pallas_call(
        _concat_body,
        grid=(pl.cdiv(V, CR),),
        in_specs=[
            pl.BlockSpec((D, CR), lambda i: (0, i)),
            pl.BlockSpec((D, CR), lambda i: (0, i)),
        ],
        out_specs=pl.BlockSpec((CR, 2 * D), lambda i: (i, 0)),
        out_shape=jax.ShapeDtypeStruct((V, 2 * D), jnp.float32),
    )(emb_lT, emb_uT)


def _gather_sc(emb_lu_hbm, idx_l_hbm, idx_u_hbm, out_l_hbm, out_u_hbm,
               idx_l_v, idx_u_v, gl, gu, sem_g, sem_w):
    wid = lax.axis_index("s") * 2 + lax.axis_index("c")
    base = wid * ROWS_W

    pltpu.sync_copy(idx_l_hbm.at[wid], idx_l_v)
    pltpu.sync_copy(idx_u_hbm.at[wid], idx_u_v)

    def g_copies(c):
        s = c % DEPTH
        return (pltpu.make_async_copy(emb_lu_hbm.at[idx_l_v.at[c]],
                                      gl.at[s], sem_g),
                pltpu.make_async_copy(emb_lu_hbm.at[idx_u_v.at[c]],
                                      gu.at[s], sem_g))

    def w_copies(c):
        s = c % DEPTH
        r = pl.ds(base + c * CHUNK, CHUNK)
        return (pltpu.make_async_copy(gl.at[s], out_l_hbm.at[r], sem_w),
                pltpu.make_async_copy(gu.at[s], out_u_hbm.at[r], sem_w))

    for c in range(DEPTH):
        for cp in g_copies(c):
            cp.start()
    for c in range(NCH):
        for cp in g_copies(c):
            cp.wait()
        for cp in w_copies(c):
            cp.start()
        if c + DEPTH < NCH:
            for cp in w_copies(c):
                cp.wait()
            for cp in g_copies(c + DEPTH):
                cp.start()
    for c in range(max(0, NCH - DEPTH), NCH):
        for cp in w_copies(c):
            cp.wait()


def _gather(emb_lu, idx_l, idx_u):
    mesh = plsc.VectorSubcoreMesh(core_axis_name="c", subcore_axis_name="s")
    return pl.kernel(
        _gather_sc,
        out_type=(jax.ShapeDtypeStruct((N, 2 * D), jnp.float32),
                  jax.ShapeDtypeStruct((N, 2 * D), jnp.float32)),
        mesh=mesh,
        scratch_types=[
            pltpu.VMEM((8, CHUNK), jnp.int32),
            pltpu.VMEM((8, CHUNK), jnp.int32),
            pltpu.VMEM((DEPTH, CHUNK, 2 * D), jnp.float32),
            pltpu.VMEM((DEPTH, CHUNK, 2 * D), jnp.float32),
            pltpu.SemaphoreType.DMA,
            pltpu.SemaphoreType.DMA,
        ],
    )(emb_lu, idx_l, idx_u)


BJ = 128


def _joint_body(tf_ref, g_l_ref, g_u_ref, et_ref, out_ref):
    et = et_ref[...].astype(jnp.bfloat16)
    iot = lax.broadcasted_iota(jnp.int32, (HOURS + 1, 1), 0)
    oh = (iot == tf_ref[0]).astype(jnp.bfloat16)
    tpart = lax.dot_general(oh, et, (((0,), (0,)), ((), ())),
                            preferred_element_type=jnp.float32)
    out_ref[...] = g_l_ref[:, 0:D] + g_u_ref[:, D:2 * D] + tpart


def _joint(tf, g_l, g_u, emb_t):
    grid = (B // BJ,)
    gspec = pl.BlockSpec((BJ * L, 2 * D), lambda i: (i, 0))
    return pl.pallas_call(
        _joint_body,
        grid=grid,
        in_specs=[
            pl.BlockSpec((1, 1, BJ * L), lambda i: (i, 0, 0)),
            gspec, gspec,
            pl.BlockSpec((HOURS + 1, D), lambda i: (0, 0)),
        ],
        out_specs=pl.BlockSpec((BJ * L, D), lambda i: (i, 0)),
        out_shape=jax.ShapeDtypeStruct((N, D), jnp.float32),
    )(tf, g_l, g_u, emb_t)


PT = 8


def _delta_body(lens_ref, coef_ref, su_ref, sl_ref, tu_ref, tl_ref,
                out_ref):
    sl = sl_ref[...]
    su = su_ref[...]
    tl = tl_ref[...]
    tu = tu_ref[...]
    a0 = (sl[0] + tl[0])[None, :]
    da = (sl[1] + tl[1])[None, :] - a0
    b0 = ((su[0] - sl[0]) * (1.0 / (SU - SL)))[None, :]
    db = ((su[1] - sl[1]) * (1.0 / (SU - SL)))[None, :] - b0
    c0 = ((tu[0] - tl[0]) * (1.0 / (TU - TL)))[None, :]
    dc = ((tu[1] - tl[1]) * (1.0 / (TU - TL)))[None, :] - c0
    zz = jnp.zeros((2, D), jnp.float32)
    vtop = jnp.concatenate([a0, b0, c0, da, db, dc, zz],
                           axis=0).astype(jnp.bfloat16)

    g = pl.program_id(0)
    lens = lens_ref[...]
    blk = coef_ref[...]
    ones = jnp.ones((1, B), jnp.float32)
    zrow = jnp.zeros((2, B), jnp.float32)
    for t in range(PT):
        p = PT * g + t
        i = p // L
        j = p % L
        m = ((i < lens) & (j < lens)).astype(jnp.float32)
        dsr = blk[2 * t:2 * t + 1]
        dtr = blk[2 * t + 1:2 * t + 2]
        ct = jnp.concatenate(
            [ones, dsr, dtr, m, m * dsr, m * dtr, zrow], axis=0
        ).astype(jnp.bfloat16)
        out_ref[t] = lax.dot_general(
            vtop, ct, (((0,), (0,)), ((), ())),
            preferred_element_type=jnp.float32)


def _delta(lens, m800T, emb_su, emb_sl, emb_tu, emb_tl):
    grid = (L * L // PT,)
    small = pl.BlockSpec((2, D), lambda i: (0, 0))
    return pl.pallas_call(
        _delta_body,
        grid=grid,
        in_specs=[
            pl.BlockSpec((1, B), lambda i: (0, 0)),
            pl.BlockSpec((2 * PT, B), lambda i: (i, 0)),
            small, small, small, small,
        ],
        out_specs=pl.BlockSpec((PT, D, B), lambda i: (i, 0, 0)),
        out_shape=jax.ShapeDtypeStruct((L * L, D, B), jnp.float32),
    )(lens, m800T, emb_su, emb_sl, emb_tu, emb_tl)


def kernel(traj, mat, traj_len, emb_t, emb_l, emb_u, emb_su, emb_sl,
           emb_tu, emb_tl):
    emb_lu = _concat(emb_l.T, emb_u.T)
    idx_l = jnp.pad(traj[:, :, 1].reshape(NW, NCH, CHUNK),
                    ((0, 0), (0, 8 - NCH), (0, 0)))
    idx_u = jnp.pad(traj[:, :, 0].reshape(NW, NCH, CHUNK),
                    ((0, 0), (0, 8 - NCH), (0, 0)))
    g_l, g_u = _gather(emb_lu, idx_l, idx_u)

    t_idx = (traj[:, :, 2] - 1) % HOURS + 1
    tf = t_idx.reshape(B // BJ, 1, BJ * L)
    joint = _joint(tf, g_l, g_u, emb_t).reshape(B, L, D)

    m800T = mat.transpose(1, 2, 3, 0).reshape(2 * L * L, B)
    delta_t = _delta(traj_len.reshape(1, B), m800T, emb_su, emb_sl,
                     emb_tu, emb_tl)
    delta = delta_t.reshape(L, L, D, B).transpose(3, 0, 1, 2)
    return (joint, delta)


# speedup vs baseline: 2.5630x; 1.2152x over previous
import jax
import jax.numpy as jnp
from jax import lax
from jax.experimental import pallas as pl
from jax.experimental.pallas import tpu as pltpu
from jax.experimental.pallas import tpu_sc as plsc

HOURS = 24 * 7
SU, SL, TU, TL = 1000.0, 0.0, 500.0, 0.0
B, L, D = 1024, 20, 64
V = 100000
N = B * L
NW = 32
ROWS_W = N // NW
CHUNK = 128
NCH = ROWS_W // CHUNK
DEPTH = 3


CR = 6400


def _concat_body(lT_ref, uT_ref, o_ref):
    o_ref[:, 0:D] = lT_ref[...].T
    o_ref[:, D:2 * D] = uT_ref[...].T


def _concat(emb_lT, emb_uT):
    return pl.pallas_call(
        _concat_body,
        grid=(pl.cdiv(V, CR),),
        in_specs=[
            pl.BlockSpec((D, CR), lambda i: (0, i)),
            pl.BlockSpec((D, CR), lambda i: (0, i)),
        ],
        out_specs=pl.BlockSpec((CR, 2 * D), lambda i: (i, 0)),
        out_shape=jax.ShapeDtypeStruct((V, 2 * D), jnp.float32),
    )(emb_lT, emb_uT)


def _gather_sc(emb_lu_hbm, idx_l_hbm, idx_u_hbm, out_l_hbm, out_u_hbm,
               idx_l_v, idx_u_v, gl, gu, sem_g, sem_w):
    wid = lax.axis_index("s") * 2 + lax.axis_index("c")
    base = wid * ROWS_W

    pltpu.sync_copy(idx_l_hbm.at[wid], idx_l_v)
    pltpu.sync_copy(idx_u_hbm.at[wid], idx_u_v)

    def g_copies(c):
        s = c % DEPTH
        return (pltpu.make_async_copy(emb_lu_hbm.at[idx_l_v.at[c]],
                                      gl.at[s], sem_g),
                pltpu.make_async_copy(emb_lu_hbm.at[idx_u_v.at[c]],
                                      gu.at[s], sem_g))

    def w_copies(c):
        s = c % DEPTH
        r = pl.ds(base + c * CHUNK, CHUNK)
        return (pltpu.make_async_copy(gl.at[s], out_l_hbm.at[r], sem_w),
                pltpu.make_async_copy(gu.at[s], out_u_hbm.at[r], sem_w))

    for c in range(DEPTH):
        for cp in g_copies(c):
            cp.start()
    for c in range(NCH):
        for cp in g_copies(c):
            cp.wait()
        for cp in w_copies(c):
            cp.start()
        if c + DEPTH < NCH:
            for cp in w_copies(c):
                cp.wait()
            for cp in g_copies(c + DEPTH):
                cp.start()
    for c in range(max(0, NCH - DEPTH), NCH):
        for cp in w_copies(c):
            cp.wait()


def _gather(emb_lu, idx_l, idx_u):
    mesh = plsc.VectorSubcoreMesh(core_axis_name="c", subcore_axis_name="s")
    return pl.kernel(
        _gather_sc,
        out_type=(jax.ShapeDtypeStruct((N, 2 * D), jnp.float32),
                  jax.ShapeDtypeStruct((N, 2 * D), jnp.float32)),
        mesh=mesh,
        scratch_types=[
            pltpu.VMEM((8, CHUNK), jnp.int32),
            pltpu.VMEM((8, CHUNK), jnp.int32),
            pltpu.VMEM((DEPTH, CHUNK, 2 * D), jnp.float32),
            pltpu.VMEM((DEPTH, CHUNK, 2 * D), jnp.float32),
            pltpu.SemaphoreType.DMA,
            pltpu.SemaphoreType.DMA,
        ],
    )(emb_lu, idx_l, idx_u)


def _joint_body(tf_ref, g_l_ref, g_u_ref, et_ref, out_ref):
    et = et_ref[...].astype(jnp.bfloat16)
    iot = lax.broadcasted_iota(jnp.int32, (HOURS + 1, 1), 0)
    oh = (iot == tf_ref[0]).astype(jnp.bfloat16)
    tpart = lax.dot_general(et, oh, (((0,), (0,)), ((), ())),
                            preferred_element_type=jnp.float32)
    out_ref[0] = g_l_ref[:, 0:D].T + g_u_ref[:, D:2 * D].T + tpart


def _joint(tfT, g_l, g_u, emb_t):
    grid = (L,)
    gspec = pl.BlockSpec((B, 2 * D), lambda i: (i, 0))
    return pl.pallas_call(
        _joint_body,
        grid=grid,
        in_specs=[
            pl.BlockSpec((1, 1, B), lambda i: (i, 0, 0)),
            gspec, gspec,
            pl.BlockSpec((HOURS + 1, D), lambda i: (0, 0)),
        ],
        out_specs=pl.BlockSpec((1, D, B), lambda i: (i, 0, 0)),
        out_shape=jax.ShapeDtypeStruct((L, D, B), jnp.float32),
    )(tfT, g_l, g_u, emb_t)


PT = 16


def _delta_body(lens_ref, coef_ref, su_ref, sl_ref, tu_ref, tl_ref,
                out_ref):
    sl = sl_ref[...]
    su = su_ref[...]
    tl = tl_ref[...]
    tu = tu_ref[...]
    a0 = (sl[0] + tl[0])[None, :]
    da = (sl[1] + tl[1])[None, :] - a0
    b0 = ((su[0] - sl[0]) * (1.0 / (SU - SL)))[None, :]
    db = ((su[1] - sl[1]) * (1.0 / (SU - SL)))[None, :] - b0
    c0 = ((tu[0] - tl[0]) * (1.0 / (TU - TL)))[None, :]
    dc = ((tu[1] - tl[1]) * (1.0 / (TU - TL)))[None, :] - c0
    zz = jnp.zeros((2, D), jnp.float32)
    vtop = jnp.concatenate([a0, b0, c0, da, db, dc, zz],
                           axis=0).astype(jnp.bfloat16)

    g = pl.program_id(0)
    lens = lens_ref[...]
    blk = coef_ref[...]
    ones = jnp.ones((1, B), jnp.float32)
    zrow = jnp.zeros((2, B), jnp.float32)
    for t in range(PT):
        p = PT * g + t
        i = p // L
        j = p % L
        m = ((i < lens) & (j < lens)).astype(jnp.float32)
        dsr = blk[2 * t:2 * t + 1]
        dtr = blk[2 * t + 1:2 * t + 2]
        ct = jnp.concatenate(
            [ones, dsr, dtr, m, m * dsr, m * dtr, zrow], axis=0
        ).astype(jnp.bfloat16)
        out_ref[t] = lax.dot_general(
            vtop, ct, (((0,), (0,)), ((), ())),
            preferred_element_type=jnp.float32)


def _delta(lens, m800T, emb_su, emb_sl, emb_tu, emb_tl):
    grid = (L * L // PT,)
    small = pl.BlockSpec((2, D), lambda i: (0, 0))
    return pl.pallas_call(
        _delta_body,
        grid=grid,
        in_specs=[
            pl.BlockSpec((1, B), lambda i: (0, 0)),
            pl.BlockSpec((2 * PT, B), lambda i: (i, 0)),
            small, small, small, small,
        ],
        out_specs=pl.BlockSpec((PT, D, B), lambda i: (i, 0, 0)),
        out_shape=jax.ShapeDtypeStruct((L * L, D, B), jnp.float32),
    )(lens, m800T, emb_su, emb_sl, emb_tu, emb_tl)


def kernel(traj, mat, traj_len, emb_t, emb_l, emb_u, emb_su, emb_sl,
           emb_tu, emb_tl):
    emb_lu = _concat(emb_l.T, emb_u.T)
    idx_l = jnp.pad(traj[:, :, 1].T.reshape(NW, NCH, CHUNK),
                    ((0, 0), (0, 8 - NCH), (0, 0)))
    idx_u = jnp.pad(traj[:, :, 0].T.reshape(NW, NCH, CHUNK),
                    ((0, 0), (0, 8 - NCH), (0, 0)))
    g_l, g_u = _gather(emb_lu, idx_l, idx_u)

    tfT = ((traj[:, :, 2].T - 1) % HOURS + 1).reshape(L, 1, B)
    joint = _joint(tfT, g_l, g_u, emb_t).transpose(2, 0, 1)

    m800T = mat.transpose(1, 2, 3, 0).reshape(2 * L * L, B)
    delta_t = _delta(traj_len.reshape(1, B), m800T, emb_su, emb_sl,
                     emb_tu, emb_tl)
    delta = delta_t.reshape(L, L, D, B).transpose(3, 0, 1, 2)
    return (joint, delta)
